# baseline (speedup 1.0000x reference)
import jax
import jax.numpy as jnp
from jax.experimental import pallas as pl
from jax.experimental.pallas import tpu as pltpu

_TILE = 64


def _route_body(eid_ref, wts_ref, slot_ref, tos_ref, ws_ref, starts_ref,
                nblocks_ref):
    nk = eid_ref.shape[0]
    num_e = starts_ref.shape[1]
    npad = tos_ref.shape[1]

    eid = eid_ref[...]
    lanes = jax.lax.broadcasted_iota(jnp.int32, (nk, num_e), 1)
    oh = eid == lanes
    oh_bf = oh.astype(jnp.bfloat16)
    oh_f = oh.astype(jnp.float32)

    row = jax.lax.broadcasted_iota(jnp.int32, (nk, nk), 0)
    col = jax.lax.broadcasted_iota(jnp.int32, (nk, nk), 1)
    lower = (col < row).astype(jnp.bfloat16)
    before = jnp.dot(lower, oh_bf, preferred_element_type=jnp.float32)
    rank = jnp.sum(before * oh_f, axis=1, keepdims=True)

    counts = jnp.sum(oh_f, axis=0, keepdims=True).astype(jnp.int32)
    aligned = ((counts + 7) // 8) * 8
    erow = jax.lax.broadcasted_iota(jnp.int32, (num_e, num_e), 0)
    ecol = jax.lax.broadcasted_iota(jnp.int32, (num_e, num_e), 1)
    tri = (erow < ecol).astype(jnp.bfloat16)
    starts_f = jnp.dot(aligned.astype(jnp.bfloat16), tri,
                       preferred_element_type=jnp.float32)
    start_of_tok = jnp.sum(starts_f * oh_f, axis=1, keepdims=True)
    slot = (start_of_tok + rank).astype(jnp.int32)

    slot_ref[...] = slot
    starts_ref[...] = starts_f.astype(jnp.int32)
    nblocks_ref[...] = (counts + (_TILE - 1)) // _TILE

    jlane = jax.lax.broadcasted_iota(jnp.int32, (nk, npad), 1)
    sel = (slot == jlane).astype(jnp.bfloat16)

    tok = jax.lax.broadcasted_iota(jnp.int32, (1, nk), 1)
    q_row = (tok // 16).astype(jnp.bfloat16)
    r_row = (tok % 16).astype(jnp.bfloat16)
    w_row = wts_ref[...]
    w_hi = w_row.astype(jnp.bfloat16)
    w_lo = (w_row - w_hi.astype(jnp.float32)).astype(jnp.bfloat16)
    dot = lambda a: jnp.dot(a, sel, preferred_element_type=jnp.float32)
    tos_ref[...] = (16.0 * dot(q_row) + dot(r_row)).astype(jnp.int32)
    ws_ref[...] = dot(w_hi) + dot(w_lo)


def _moe_body(starts_ref, nblocks_ref, xs_ref, ws_ref, wg_ref, wu_ref,
              wd_ref, out_ref):
    e = pl.program_id(0)
    start = starts_ref[e]
    nb = nblocks_ref[e]
    wg = wg_ref[0].astype(jnp.bfloat16)
    wu = wu_ref[0].astype(jnp.bfloat16)
    wd = wd_ref[0].astype(jnp.bfloat16)

    def tile(k, carry):
        offs = pl.multiple_of(start + k * _TILE, 8)
        x = xs_ref[pl.ds(offs, _TILE), :].astype(jnp.bfloat16)
        g = jnp.dot(x, wg, preferred_element_type=jnp.float32)
        u = jnp.dot(x, wu, preferred_element_type=jnp.float32)
        a = ((g * jax.nn.sigmoid(g)) * u).astype(jnp.bfloat16)
        o = jnp.dot(a, wd, preferred_element_type=jnp.float32)
        w = ws_ref[pl.ds(offs, _TILE), :]
        out_ref[pl.ds(offs, _TILE), :] = o * w
        return carry

    jax.lax.fori_loop(0, nb, tile, 0)


def kernel(hidden_states, top_k_indices, top_k_weights, Wg, Wu, Wd):
    N, D = hidden_states.shape
    E, _, H = Wg.shape
    K = top_k_indices.shape[1]
    NK = N * K

    npad = NK + 8 * E + 4 * _TILE
    npad = ((npad + 255) // 256) * 256

    eid = top_k_indices.reshape(NK, 1).astype(jnp.int32)
    wts = top_k_weights.reshape(1, NK).astype(jnp.float32)

    slot, tos, ws, starts, nblocks = pl.pallas_call(
        _route_body,
        out_shape=(
            jax.ShapeDtypeStruct((NK, 1), jnp.int32),
            jax.ShapeDtypeStruct((1, npad), jnp.int32),
            jax.ShapeDtypeStruct((1, npad), jnp.float32),
            jax.ShapeDtypeStruct((1, E), jnp.int32),
            jax.ShapeDtypeStruct((1, E), jnp.int32),
        ),
    )(eid, wts)

    if K > 1:
        hs = hidden_states[
            jnp.repeat(jnp.arange(N, dtype=jnp.int32), K)]
    else:
        hs = hidden_states
    xs = hs[tos.reshape(npad)]
    ws_col = ws.reshape(npad, 1)

    ys = pl.pallas_call(
        _moe_body,
        grid_spec=pltpu.PrefetchScalarGridSpec(
            num_scalar_prefetch=2,
            grid=(E,),
            in_specs=[
                pl.BlockSpec((npad, D), lambda e, s, nb: (0, 0)),
                pl.BlockSpec((npad, 1), lambda e, s, nb: (0, 0)),
                pl.BlockSpec((1, D, H), lambda e, s, nb: (e, 0, 0)),
                pl.BlockSpec((1, D, H), lambda e, s, nb: (e, 0, 0)),
                pl.BlockSpec((1, H, D), lambda e, s, nb: (e, 0, 0)),
            ],
            out_specs=pl.BlockSpec((npad, D), lambda e, s, nb: (0, 0)),
        ),
        out_shape=jax.ShapeDtypeStruct((npad, D), jnp.float32),
        compiler_params=pltpu.CompilerParams(
            dimension_semantics=("arbitrary",)),
    )(starts.reshape(E), nblocks.reshape(E), xs, ws_col, Wg, Wu, Wd)

    slot2 = slot.reshape(N, K)
    out = ys[slot2[:, 0]]
    for k in range(1, K):
        out = out + ys[slot2[:, k]]
    return out
